# Initial kernel scaffold; baseline (speedup 1.0000x reference)
import jax
import jax.numpy as jnp
from jax.experimental import pallas as pl


def kernel(atomic_numbers, table, W, b):
    raise NotImplementedError("write your pallas kernel here")


# speedup vs baseline: 9.2764x; 9.2764x over previous
import functools

import jax
import jax.numpy as jnp
import numpy as np
from jax import lax
from jax.experimental import pallas as pl
from jax.experimental.pallas import tpu as pltpu
from jax.experimental.pallas import tpu_sc as plsc

_ATOMIC_MASSES = [0.0, 1.008, 4.003, 6.941, 9.012, 10.81, 12.01, 14.01, 16.0, 19.0, 20.18, 22.99, 24.31, 26.98, 28.09, 30.97, 32.07, 35.45, 39.95, 39.1, 40.08, 44.96, 47.87, 50.94, 52.0, 54.94, 55.85, 58.93, 58.69, 63.55, 65.38, 69.72, 72.63, 74.92, 78.97, 79.9, 83.8, 85.47, 87.62, 88.91, 91.22, 92.91, 95.95, 98.0, 101.1, 102.9, 106.4, 107.9, 112.4, 114.8, 118.7, 121.8, 127.6, 126.9, 131.3, 132.9, 137.3, 138.9, 140.1, 140.9, 144.2, 145.0, 150.4, 152.0, 157.3, 158.9, 162.5, 164.9, 167.3, 168.9, 173.0, 175.0, 178.5, 180.9, 183.8, 186.2, 190.2, 192.2, 195.1, 197.0, 200.6, 204.4, 207.2, 209.0, 209.0, 210.0, 222.0, 223.0, 226.0, 227.0, 232.0, 231.0, 238.0, 237.0, 244.0, 243.0, 247.0, 247.0, 251.0, 252.0, 257.0, 258.0, 259.0, 262.0, 267.0, 270.0, 269.0, 270.0, 270.0, 278.0, 281.0, 281.0, 285.0, 286.0, 289.0, 289.0, 293.0, 293.0, 294.0]
_ATOMIC_RADII = [0.0, 1.2, 1.4, 1.82, 1.53, 1.92, 1.7, 1.55, 1.52, 1.47, 1.54, 2.27, 1.73, 1.84, 2.1, 1.8, 1.8, 1.75, 1.88, 2.75, 2.31, 2.11, 2.0, 2.0, 2.0, 2.0, 2.0, 2.0, 1.63, 1.4, 1.39, 1.87, 2.11, 1.85, 1.9, 1.85, 2.02, 3.03, 2.49, 2.0, 2.0, 2.0, 2.0, 2.0, 2.0, 2.0, 1.63, 1.72, 1.58, 1.93, 2.17, 2.06, 2.06, 1.98, 2.16, 3.43, 2.68, 2.0, 2.0, 2.0, 2.0, 2.0, 2.0, 2.0, 2.0, 2.0, 2.0, 2.0, 2.0, 2.0, 2.0, 2.0, 2.0, 2.0, 2.0, 2.0, 2.0, 2.0, 1.75, 1.66, 1.55, 1.96, 2.02, 2.07, 1.97, 2.02, 2.2, 3.48, 2.83, 2.0, 2.0, 2.0, 1.86, 2.0, 2.0, 2.0, 2.0, 2.0, 2.0, 2.0, 2.0, 2.0, 2.0, 2.0, 2.0, 2.0, 2.0, 2.0, 2.0, 2.0, 2.0, 2.0, 2.0, 2.0, 2.0, 2.0, 2.0, 2.0, 2.0]
_ELECTRONEGATIVITIES = [0.0, 2.2, 0.0, 0.98, 1.57, 2.04, 2.55, 3.04, 3.44, 3.98, 0.0, 0.93, 1.31, 1.61, 1.9, 2.19, 2.58, 3.16, 0.0, 0.82, 1.0, 1.36, 1.54, 1.63, 1.66, 1.55, 1.83, 1.88, 1.91, 1.9, 1.65, 1.81, 2.01, 2.18, 2.55, 2.96, 3.0, 0.82, 0.95, 1.22, 1.33, 1.6, 2.16, 1.9, 2.2, 2.28, 2.2, 1.93, 1.69, 1.78, 1.96, 2.05, 2.1, 2.66, 2.6, 0.79, 0.89, 1.1, 1.12, 1.13, 1.14, 1.13, 1.17, 1.2, 1.2, 1.22, 1.23, 1.24, 1.25, 1.1, 1.27, 1.3, 1.5, 2.36, 1.9, 2.2, 2.2, 2.28, 2.54, 2.0, 1.62, 1.87, 2.33, 2.02, 2.0, 2.2, 2.2, 0.7, 0.9, 1.1, 1.3, 1.5, 1.38, 1.36, 1.28, 1.3, 1.3, 1.3, 1.3, 1.3, 1.3, 1.3, 1.3, 1.3, 1.3, 1.3, 1.3, 1.3, 1.3, 1.3, 1.3, 1.3, 1.3, 1.3, 1.3, 1.3, 1.3, 1.3, 1.3]
_IONIZATION_ENERGIES = [0.0, 13.6, 24.59, 5.39, 9.32, 8.3, 11.26, 14.53, 13.62, 17.42, 21.56, 5.14, 7.65, 5.99, 8.15, 10.49, 10.36, 12.97, 15.76, 4.34, 6.11, 6.56, 6.83, 6.75, 6.77, 7.43, 7.9, 7.88, 7.64, 7.73, 9.39, 6.0, 7.9, 9.79, 9.75, 11.81, 14.0, 4.18, 5.69, 6.22, 6.63, 6.76, 7.09, 7.28, 7.36, 7.46, 8.34, 7.58, 8.99, 5.79, 7.34, 8.64, 9.01, 10.45, 12.13, 3.89, 5.21, 5.58, 5.54, 5.47, 5.53, 5.58, 5.64, 5.67, 6.15, 5.86, 5.94, 6.02, 6.11, 6.18, 6.25, 5.43, 6.83, 7.55, 7.86, 7.83, 8.44, 8.97, 8.96, 9.23, 10.44, 6.11, 7.42, 7.29, 8.42, 9.3, 10.75, 4.07, 5.28, 5.17, 6.31, 5.89, 6.19, 6.27, 6.03, 5.97, 6.02, 6.2, 6.28, 6.42, 6.5, 6.58, 6.65, 4.9, 6.0, 6.0, 6.0, 6.0, 6.0, 6.0, 6.0, 6.0, 6.0, 6.0, 6.0, 6.0, 6.0, 6.0, 6.0]

_PROPS = np.zeros((119, 4), dtype=np.float32)
_PROPS[: len(_ATOMIC_MASSES), 0] = _ATOMIC_MASSES
_PROPS[: len(_ATOMIC_RADII), 1] = _ATOMIC_RADII
_PROPS[: len(_ELECTRONEGATIVITIES), 2] = _ELECTRONEGATIVITIES
_PROPS[: len(_IONIZATION_ENERGIES), 3] = _IONIZATION_ENERGIES

_N = 100000
_D = 128
_NW = 32
_CHUNK = 128
_K = 25
_PER_W = _CHUNK * _K
_NPAD = _NW * _PER_W


def _fuse_body(ct_ref, w_ref, b_ref, out_ref):
    out_ref[...] = (
        jnp.dot(ct_ref[...], w_ref[...], preferred_element_type=jnp.float32)
        + b_ref[...]
    )


def _fused_table(table, W, b):
    ct = jnp.zeros((128, 256), dtype=jnp.float32)
    ct = ct.at[:119, :_D].set(table)
    ct = ct.at[:119, _D : _D + 4].set(_PROPS)
    w_pad = jnp.zeros((256, _D), dtype=jnp.float32).at[: _D + 4, :].set(W)
    return pl.pallas_call(
        _fuse_body,
        out_shape=jax.ShapeDtypeStruct((128, _D), jnp.float32),
    )(ct, w_pad, b.reshape(1, _D))


_sc_mesh = plsc.VectorSubcoreMesh(core_axis_name="c", subcore_axis_name="s")


@functools.partial(
    pl.kernel,
    mesh=_sc_mesh,
    out_type=jax.ShapeDtypeStruct((_NPAD, _D), jnp.float32),
    scratch_types=[
        pltpu.VMEM((_K, _CHUNK), jnp.int32),
        pltpu.VMEM((_CHUNK, _D), jnp.float32),
        pltpu.SemaphoreType.DMA,
    ],
)
def _sc_gather(idx_hbm, fused_hbm, out_hbm, idx_v, rows_v, sem):
    wid = lax.axis_index("s") * 2 + lax.axis_index("c")
    base = wid * _PER_W
    pltpu.sync_copy(idx_hbm.at[wid], idx_v)

    def body(j, carry):
        pltpu.async_copy(fused_hbm.at[idx_v.at[j]], rows_v, sem).wait()
        pltpu.sync_copy(rows_v, out_hbm.at[pl.ds(base + j * _CHUNK, _CHUNK)])
        return carry

    lax.fori_loop(0, _K, body, 0)


def kernel(atomic_numbers, table, W, b):
    fused = _fused_table(table, W, b)
    an = atomic_numbers.astype(jnp.int32)
    an_pad = jnp.zeros((_NPAD,), jnp.int32).at[:_N].set(an)
    idx3 = an_pad.reshape(_NW, _K, _CHUNK)
    out = _sc_gather(idx3, fused)
    return out[:_N]


# speedup vs baseline: 9.7315x; 1.0491x over previous
import functools

import jax
import jax.numpy as jnp
import numpy as np
from jax import lax
from jax.experimental import pallas as pl
from jax.experimental.pallas import tpu as pltpu
from jax.experimental.pallas import tpu_sc as plsc

_ATOMIC_MASSES = [0.0, 1.008, 4.003, 6.941, 9.012, 10.81, 12.01, 14.01, 16.0, 19.0, 20.18, 22.99, 24.31, 26.98, 28.09, 30.97, 32.07, 35.45, 39.95, 39.1, 40.08, 44.96, 47.87, 50.94, 52.0, 54.94, 55.85, 58.93, 58.69, 63.55, 65.38, 69.72, 72.63, 74.92, 78.97, 79.9, 83.8, 85.47, 87.62, 88.91, 91.22, 92.91, 95.95, 98.0, 101.1, 102.9, 106.4, 107.9, 112.4, 114.8, 118.7, 121.8, 127.6, 126.9, 131.3, 132.9, 137.3, 138.9, 140.1, 140.9, 144.2, 145.0, 150.4, 152.0, 157.3, 158.9, 162.5, 164.9, 167.3, 168.9, 173.0, 175.0, 178.5, 180.9, 183.8, 186.2, 190.2, 192.2, 195.1, 197.0, 200.6, 204.4, 207.2, 209.0, 209.0, 210.0, 222.0, 223.0, 226.0, 227.0, 232.0, 231.0, 238.0, 237.0, 244.0, 243.0, 247.0, 247.0, 251.0, 252.0, 257.0, 258.0, 259.0, 262.0, 267.0, 270.0, 269.0, 270.0, 270.0, 278.0, 281.0, 281.0, 285.0, 286.0, 289.0, 289.0, 293.0, 293.0, 294.0]
_ATOMIC_RADII = [0.0, 1.2, 1.4, 1.82, 1.53, 1.92, 1.7, 1.55, 1.52, 1.47, 1.54, 2.27, 1.73, 1.84, 2.1, 1.8, 1.8, 1.75, 1.88, 2.75, 2.31, 2.11, 2.0, 2.0, 2.0, 2.0, 2.0, 2.0, 1.63, 1.4, 1.39, 1.87, 2.11, 1.85, 1.9, 1.85, 2.02, 3.03, 2.49, 2.0, 2.0, 2.0, 2.0, 2.0, 2.0, 2.0, 1.63, 1.72, 1.58, 1.93, 2.17, 2.06, 2.06, 1.98, 2.16, 3.43, 2.68, 2.0, 2.0, 2.0, 2.0, 2.0, 2.0, 2.0, 2.0, 2.0, 2.0, 2.0, 2.0, 2.0, 2.0, 2.0, 2.0, 2.0, 2.0, 2.0, 2.0, 2.0, 1.75, 1.66, 1.55, 1.96, 2.02, 2.07, 1.97, 2.02, 2.2, 3.48, 2.83, 2.0, 2.0, 2.0, 1.86, 2.0, 2.0, 2.0, 2.0, 2.0, 2.0, 2.0, 2.0, 2.0, 2.0, 2.0, 2.0, 2.0, 2.0, 2.0, 2.0, 2.0, 2.0, 2.0, 2.0, 2.0, 2.0, 2.0, 2.0, 2.0, 2.0]
_ELECTRONEGATIVITIES = [0.0, 2.2, 0.0, 0.98, 1.57, 2.04, 2.55, 3.04, 3.44, 3.98, 0.0, 0.93, 1.31, 1.61, 1.9, 2.19, 2.58, 3.16, 0.0, 0.82, 1.0, 1.36, 1.54, 1.63, 1.66, 1.55, 1.83, 1.88, 1.91, 1.9, 1.65, 1.81, 2.01, 2.18, 2.55, 2.96, 3.0, 0.82, 0.95, 1.22, 1.33, 1.6, 2.16, 1.9, 2.2, 2.28, 2.2, 1.93, 1.69, 1.78, 1.96, 2.05, 2.1, 2.66, 2.6, 0.79, 0.89, 1.1, 1.12, 1.13, 1.14, 1.13, 1.17, 1.2, 1.2, 1.22, 1.23, 1.24, 1.25, 1.1, 1.27, 1.3, 1.5, 2.36, 1.9, 2.2, 2.2, 2.28, 2.54, 2.0, 1.62, 1.87, 2.33, 2.02, 2.0, 2.2, 2.2, 0.7, 0.9, 1.1, 1.3, 1.5, 1.38, 1.36, 1.28, 1.3, 1.3, 1.3, 1.3, 1.3, 1.3, 1.3, 1.3, 1.3, 1.3, 1.3, 1.3, 1.3, 1.3, 1.3, 1.3, 1.3, 1.3, 1.3, 1.3, 1.3, 1.3, 1.3, 1.3]
_IONIZATION_ENERGIES = [0.0, 13.6, 24.59, 5.39, 9.32, 8.3, 11.26, 14.53, 13.62, 17.42, 21.56, 5.14, 7.65, 5.99, 8.15, 10.49, 10.36, 12.97, 15.76, 4.34, 6.11, 6.56, 6.83, 6.75, 6.77, 7.43, 7.9, 7.88, 7.64, 7.73, 9.39, 6.0, 7.9, 9.79, 9.75, 11.81, 14.0, 4.18, 5.69, 6.22, 6.63, 6.76, 7.09, 7.28, 7.36, 7.46, 8.34, 7.58, 8.99, 5.79, 7.34, 8.64, 9.01, 10.45, 12.13, 3.89, 5.21, 5.58, 5.54, 5.47, 5.53, 5.58, 5.64, 5.67, 6.15, 5.86, 5.94, 6.02, 6.11, 6.18, 6.25, 5.43, 6.83, 7.55, 7.86, 7.83, 8.44, 8.97, 8.96, 9.23, 10.44, 6.11, 7.42, 7.29, 8.42, 9.3, 10.75, 4.07, 5.28, 5.17, 6.31, 5.89, 6.19, 6.27, 6.03, 5.97, 6.02, 6.2, 6.28, 6.42, 6.5, 6.58, 6.65, 4.9, 6.0, 6.0, 6.0, 6.0, 6.0, 6.0, 6.0, 6.0, 6.0, 6.0, 6.0, 6.0, 6.0, 6.0, 6.0]

_PROPS = np.zeros((119, 4), dtype=np.float32)
_PROPS[: len(_ATOMIC_MASSES), 0] = _ATOMIC_MASSES
_PROPS[: len(_ATOMIC_RADII), 1] = _ATOMIC_RADII
_PROPS[: len(_ELECTRONEGATIVITIES), 2] = _ELECTRONEGATIVITIES
_PROPS[: len(_IONIZATION_ENERGIES), 3] = _IONIZATION_ENERGIES

_N = 100000
_D = 128
_NW = 32
_CHUNK = 128
_K = 25
_PER_W = _CHUNK * _K
_NPAD = _NW * _PER_W


def _fuse_body(ct_ref, w_ref, b_ref, out_ref):
    out_ref[...] = (
        jnp.dot(ct_ref[...], w_ref[...], preferred_element_type=jnp.float32)
        + b_ref[...]
    )


def _fused_table(table, W, b):
    ct = jnp.zeros((128, 256), dtype=jnp.float32)
    ct = ct.at[:119, :_D].set(table)
    ct = ct.at[:119, _D : _D + 4].set(_PROPS)
    w_pad = jnp.zeros((256, _D), dtype=jnp.float32).at[: _D + 4, :].set(W)
    return pl.pallas_call(
        _fuse_body,
        out_shape=jax.ShapeDtypeStruct((128, _D), jnp.float32),
    )(ct, w_pad, b.reshape(1, _D))


_sc_mesh = plsc.VectorSubcoreMesh(core_axis_name="c", subcore_axis_name="s")

_NBUF = 5
_KOUT = _K // _NBUF


@functools.partial(
    pl.kernel,
    mesh=_sc_mesh,
    out_type=jax.ShapeDtypeStruct((_NPAD, _D), jnp.float32),
    scratch_types=(
        [pltpu.VMEM((_K, _CHUNK), jnp.int32)]
        + [pltpu.VMEM((_CHUNK, _D), jnp.float32)] * _NBUF
        + [pltpu.SemaphoreType.DMA] * (2 * _NBUF)
    ),
)
def _sc_gather(idx_hbm, fused_hbm, out_hbm, idx_v, *scratch):
    bufs = scratch[:_NBUF]
    gsems = scratch[_NBUF : 2 * _NBUF]
    ssems = scratch[2 * _NBUF :]
    wid = lax.axis_index("s") * 2 + lax.axis_index("c")
    base = wid * _PER_W
    pltpu.sync_copy(idx_hbm.at[wid], idx_v)
    for b in range(_NBUF):
        pltpu.async_copy(fused_hbm.at[idx_v.at[b]], bufs[b], gsems[b])

    def outer(g, carry):
        for b in range(_NBUF):
            j = g * _NBUF + b
            pltpu.make_async_copy(
                fused_hbm.at[idx_v.at[j]], bufs[b], gsems[b]
            ).wait()
            sc = pltpu.make_async_copy(
                bufs[b], out_hbm.at[pl.ds(base + j * _CHUNK, _CHUNK)], ssems[b]
            )
            sc.start()
            sc.wait()

            @pl.when(g < _KOUT - 1)
            def _():
                pltpu.async_copy(
                    fused_hbm.at[idx_v.at[j + _NBUF]], bufs[b], gsems[b]
                )

        return carry

    lax.fori_loop(0, _KOUT, outer, 0)


def kernel(atomic_numbers, table, W, b):
    fused = _fused_table(table, W, b)
    an = atomic_numbers.astype(jnp.int32)
    an_pad = jnp.zeros((_NPAD,), jnp.int32).at[:_N].set(an)
    idx3 = an_pad.reshape(_NW, _K, _CHUNK)
    out = _sc_gather(idx3, fused)
    return out[:_N]


# speedup vs baseline: 33.2154x; 3.4132x over previous
import functools

import jax
import jax.numpy as jnp
import numpy as np
from jax import lax
from jax.experimental import pallas as pl
from jax.experimental.pallas import tpu as pltpu
from jax.experimental.pallas import tpu_sc as plsc

_ATOMIC_MASSES = [0.0, 1.008, 4.003, 6.941, 9.012, 10.81, 12.01, 14.01, 16.0, 19.0, 20.18, 22.99, 24.31, 26.98, 28.09, 30.97, 32.07, 35.45, 39.95, 39.1, 40.08, 44.96, 47.87, 50.94, 52.0, 54.94, 55.85, 58.93, 58.69, 63.55, 65.38, 69.72, 72.63, 74.92, 78.97, 79.9, 83.8, 85.47, 87.62, 88.91, 91.22, 92.91, 95.95, 98.0, 101.1, 102.9, 106.4, 107.9, 112.4, 114.8, 118.7, 121.8, 127.6, 126.9, 131.3, 132.9, 137.3, 138.9, 140.1, 140.9, 144.2, 145.0, 150.4, 152.0, 157.3, 158.9, 162.5, 164.9, 167.3, 168.9, 173.0, 175.0, 178.5, 180.9, 183.8, 186.2, 190.2, 192.2, 195.1, 197.0, 200.6, 204.4, 207.2, 209.0, 209.0, 210.0, 222.0, 223.0, 226.0, 227.0, 232.0, 231.0, 238.0, 237.0, 244.0, 243.0, 247.0, 247.0, 251.0, 252.0, 257.0, 258.0, 259.0, 262.0, 267.0, 270.0, 269.0, 270.0, 270.0, 278.0, 281.0, 281.0, 285.0, 286.0, 289.0, 289.0, 293.0, 293.0, 294.0]
_ATOMIC_RADII = [0.0, 1.2, 1.4, 1.82, 1.53, 1.92, 1.7, 1.55, 1.52, 1.47, 1.54, 2.27, 1.73, 1.84, 2.1, 1.8, 1.8, 1.75, 1.88, 2.75, 2.31, 2.11, 2.0, 2.0, 2.0, 2.0, 2.0, 2.0, 1.63, 1.4, 1.39, 1.87, 2.11, 1.85, 1.9, 1.85, 2.02, 3.03, 2.49, 2.0, 2.0, 2.0, 2.0, 2.0, 2.0, 2.0, 1.63, 1.72, 1.58, 1.93, 2.17, 2.06, 2.06, 1.98, 2.16, 3.43, 2.68, 2.0, 2.0, 2.0, 2.0, 2.0, 2.0, 2.0, 2.0, 2.0, 2.0, 2.0, 2.0, 2.0, 2.0, 2.0, 2.0, 2.0, 2.0, 2.0, 2.0, 2.0, 1.75, 1.66, 1.55, 1.96, 2.02, 2.07, 1.97, 2.02, 2.2, 3.48, 2.83, 2.0, 2.0, 2.0, 1.86, 2.0, 2.0, 2.0, 2.0, 2.0, 2.0, 2.0, 2.0, 2.0, 2.0, 2.0, 2.0, 2.0, 2.0, 2.0, 2.0, 2.0, 2.0, 2.0, 2.0, 2.0, 2.0, 2.0, 2.0, 2.0, 2.0]
_ELECTRONEGATIVITIES = [0.0, 2.2, 0.0, 0.98, 1.57, 2.04, 2.55, 3.04, 3.44, 3.98, 0.0, 0.93, 1.31, 1.61, 1.9, 2.19, 2.58, 3.16, 0.0, 0.82, 1.0, 1.36, 1.54, 1.63, 1.66, 1.55, 1.83, 1.88, 1.91, 1.9, 1.65, 1.81, 2.01, 2.18, 2.55, 2.96, 3.0, 0.82, 0.95, 1.22, 1.33, 1.6, 2.16, 1.9, 2.2, 2.28, 2.2, 1.93, 1.69, 1.78, 1.96, 2.05, 2.1, 2.66, 2.6, 0.79, 0.89, 1.1, 1.12, 1.13, 1.14, 1.13, 1.17, 1.2, 1.2, 1.22, 1.23, 1.24, 1.25, 1.1, 1.27, 1.3, 1.5, 2.36, 1.9, 2.2, 2.2, 2.28, 2.54, 2.0, 1.62, 1.87, 2.33, 2.02, 2.0, 2.2, 2.2, 0.7, 0.9, 1.1, 1.3, 1.5, 1.38, 1.36, 1.28, 1.3, 1.3, 1.3, 1.3, 1.3, 1.3, 1.3, 1.3, 1.3, 1.3, 1.3, 1.3, 1.3, 1.3, 1.3, 1.3, 1.3, 1.3, 1.3, 1.3, 1.3, 1.3, 1.3, 1.3]
_IONIZATION_ENERGIES = [0.0, 13.6, 24.59, 5.39, 9.32, 8.3, 11.26, 14.53, 13.62, 17.42, 21.56, 5.14, 7.65, 5.99, 8.15, 10.49, 10.36, 12.97, 15.76, 4.34, 6.11, 6.56, 6.83, 6.75, 6.77, 7.43, 7.9, 7.88, 7.64, 7.73, 9.39, 6.0, 7.9, 9.79, 9.75, 11.81, 14.0, 4.18, 5.69, 6.22, 6.63, 6.76, 7.09, 7.28, 7.36, 7.46, 8.34, 7.58, 8.99, 5.79, 7.34, 8.64, 9.01, 10.45, 12.13, 3.89, 5.21, 5.58, 5.54, 5.47, 5.53, 5.58, 5.64, 5.67, 6.15, 5.86, 5.94, 6.02, 6.11, 6.18, 6.25, 5.43, 6.83, 7.55, 7.86, 7.83, 8.44, 8.97, 8.96, 9.23, 10.44, 6.11, 7.42, 7.29, 8.42, 9.3, 10.75, 4.07, 5.28, 5.17, 6.31, 5.89, 6.19, 6.27, 6.03, 5.97, 6.02, 6.2, 6.28, 6.42, 6.5, 6.58, 6.65, 4.9, 6.0, 6.0, 6.0, 6.0, 6.0, 6.0, 6.0, 6.0, 6.0, 6.0, 6.0, 6.0, 6.0, 6.0, 6.0]

_PROPS = np.zeros((119, 4), dtype=np.float32)
_PROPS[: len(_ATOMIC_MASSES), 0] = _ATOMIC_MASSES
_PROPS[: len(_ATOMIC_RADII), 1] = _ATOMIC_RADII
_PROPS[: len(_ELECTRONEGATIVITIES), 2] = _ELECTRONEGATIVITIES
_PROPS[: len(_IONIZATION_ENERGIES), 3] = _IONIZATION_ENERGIES

_N = 100000
_D = 128
_NW = 32
_CHUNK = 128
_K = 25
_PER_W = _CHUNK * _K
_NPAD = _NW * _PER_W


def _fuse_body(ct_ref, w_ref, b_ref, out_ref):
    out_ref[...] = (
        jnp.dot(ct_ref[...], w_ref[...], preferred_element_type=jnp.float32)
        + b_ref[...]
    )


def _fused_table(table, W, b):
    ct = jnp.zeros((128, 256), dtype=jnp.float32)
    ct = ct.at[:119, :_D].set(table)
    ct = ct.at[:119, _D : _D + 4].set(_PROPS)
    w_pad = jnp.zeros((256, _D), dtype=jnp.float32).at[: _D + 4, :].set(W)
    return pl.pallas_call(
        _fuse_body,
        out_shape=jax.ShapeDtypeStruct((128, _D), jnp.float32),
    )(ct, w_pad, b.reshape(1, _D))


_sc_mesh = plsc.VectorSubcoreMesh(core_axis_name="c", subcore_axis_name="s")

_NBUF = 5
_KOUT = _K // _NBUF


@functools.partial(
    pl.kernel,
    mesh=_sc_mesh,
    out_type=jax.ShapeDtypeStruct((_NPAD, _D), jnp.float32),
    scratch_types=(
        [pltpu.VMEM((_K, _CHUNK), jnp.int32)]
        + [pltpu.VMEM_SHARED((128, _D), jnp.float32)]
        + [pltpu.VMEM((_CHUNK, _D), jnp.float32)] * _NBUF
        + [pltpu.SemaphoreType.DMA] * (2 * _NBUF)
    ),
)
def _sc_gather(idx_hbm, fused_hbm, out_hbm, idx_v, shared_tab, *scratch):
    bufs = scratch[:_NBUF]
    gsems = scratch[_NBUF : 2 * _NBUF]
    ssems = scratch[2 * _NBUF :]
    sid = lax.axis_index("s")
    wid = sid * 2 + lax.axis_index("c")
    base = wid * _PER_W
    pltpu.sync_copy(idx_hbm.at[wid], idx_v)

    @pl.when(sid == 0)
    def _():
        pltpu.sync_copy(fused_hbm, shared_tab)

    plsc.subcore_barrier()
    for b in range(_NBUF):
        pltpu.async_copy(shared_tab.at[idx_v.at[b]], bufs[b], gsems[b])

    def outer(g, carry):
        for b in range(_NBUF):
            j = g * _NBUF + b
            pltpu.make_async_copy(
                shared_tab.at[idx_v.at[j]], bufs[b], gsems[b]
            ).wait()
            sc = pltpu.make_async_copy(
                bufs[b], out_hbm.at[pl.ds(base + j * _CHUNK, _CHUNK)], ssems[b]
            )
            sc.start()
            sc.wait()

            @pl.when(g < _KOUT - 1)
            def _():
                pltpu.async_copy(
                    shared_tab.at[idx_v.at[j + _NBUF]], bufs[b], gsems[b]
                )

        return carry

    lax.fori_loop(0, _KOUT, outer, 0)


def kernel(atomic_numbers, table, W, b):
    fused = _fused_table(table, W, b)
    an = atomic_numbers.astype(jnp.int32)
    an_pad = jnp.zeros((_NPAD,), jnp.int32).at[:_N].set(an)
    idx3 = an_pad.reshape(_NW, _K, _CHUNK)
    out = _sc_gather(idx3, fused)
    return out[:_N]


# speedup vs baseline: 53.5527x; 1.6123x over previous
import functools

import jax
import jax.numpy as jnp
import numpy as np
from jax import lax
from jax.experimental import pallas as pl
from jax.experimental.pallas import tpu as pltpu
from jax.experimental.pallas import tpu_sc as plsc

_ATOMIC_MASSES = [0.0, 1.008, 4.003, 6.941, 9.012, 10.81, 12.01, 14.01, 16.0, 19.0, 20.18, 22.99, 24.31, 26.98, 28.09, 30.97, 32.07, 35.45, 39.95, 39.1, 40.08, 44.96, 47.87, 50.94, 52.0, 54.94, 55.85, 58.93, 58.69, 63.55, 65.38, 69.72, 72.63, 74.92, 78.97, 79.9, 83.8, 85.47, 87.62, 88.91, 91.22, 92.91, 95.95, 98.0, 101.1, 102.9, 106.4, 107.9, 112.4, 114.8, 118.7, 121.8, 127.6, 126.9, 131.3, 132.9, 137.3, 138.9, 140.1, 140.9, 144.2, 145.0, 150.4, 152.0, 157.3, 158.9, 162.5, 164.9, 167.3, 168.9, 173.0, 175.0, 178.5, 180.9, 183.8, 186.2, 190.2, 192.2, 195.1, 197.0, 200.6, 204.4, 207.2, 209.0, 209.0, 210.0, 222.0, 223.0, 226.0, 227.0, 232.0, 231.0, 238.0, 237.0, 244.0, 243.0, 247.0, 247.0, 251.0, 252.0, 257.0, 258.0, 259.0, 262.0, 267.0, 270.0, 269.0, 270.0, 270.0, 278.0, 281.0, 281.0, 285.0, 286.0, 289.0, 289.0, 293.0, 293.0, 294.0]
_ATOMIC_RADII = [0.0, 1.2, 1.4, 1.82, 1.53, 1.92, 1.7, 1.55, 1.52, 1.47, 1.54, 2.27, 1.73, 1.84, 2.1, 1.8, 1.8, 1.75, 1.88, 2.75, 2.31, 2.11, 2.0, 2.0, 2.0, 2.0, 2.0, 2.0, 1.63, 1.4, 1.39, 1.87, 2.11, 1.85, 1.9, 1.85, 2.02, 3.03, 2.49, 2.0, 2.0, 2.0, 2.0, 2.0, 2.0, 2.0, 1.63, 1.72, 1.58, 1.93, 2.17, 2.06, 2.06, 1.98, 2.16, 3.43, 2.68, 2.0, 2.0, 2.0, 2.0, 2.0, 2.0, 2.0, 2.0, 2.0, 2.0, 2.0, 2.0, 2.0, 2.0, 2.0, 2.0, 2.0, 2.0, 2.0, 2.0, 2.0, 1.75, 1.66, 1.55, 1.96, 2.02, 2.07, 1.97, 2.02, 2.2, 3.48, 2.83, 2.0, 2.0, 2.0, 1.86, 2.0, 2.0, 2.0, 2.0, 2.0, 2.0, 2.0, 2.0, 2.0, 2.0, 2.0, 2.0, 2.0, 2.0, 2.0, 2.0, 2.0, 2.0, 2.0, 2.0, 2.0, 2.0, 2.0, 2.0, 2.0, 2.0]
_ELECTRONEGATIVITIES = [0.0, 2.2, 0.0, 0.98, 1.57, 2.04, 2.55, 3.04, 3.44, 3.98, 0.0, 0.93, 1.31, 1.61, 1.9, 2.19, 2.58, 3.16, 0.0, 0.82, 1.0, 1.36, 1.54, 1.63, 1.66, 1.55, 1.83, 1.88, 1.91, 1.9, 1.65, 1.81, 2.01, 2.18, 2.55, 2.96, 3.0, 0.82, 0.95, 1.22, 1.33, 1.6, 2.16, 1.9, 2.2, 2.28, 2.2, 1.93, 1.69, 1.78, 1.96, 2.05, 2.1, 2.66, 2.6, 0.79, 0.89, 1.1, 1.12, 1.13, 1.14, 1.13, 1.17, 1.2, 1.2, 1.22, 1.23, 1.24, 1.25, 1.1, 1.27, 1.3, 1.5, 2.36, 1.9, 2.2, 2.2, 2.28, 2.54, 2.0, 1.62, 1.87, 2.33, 2.02, 2.0, 2.2, 2.2, 0.7, 0.9, 1.1, 1.3, 1.5, 1.38, 1.36, 1.28, 1.3, 1.3, 1.3, 1.3, 1.3, 1.3, 1.3, 1.3, 1.3, 1.3, 1.3, 1.3, 1.3, 1.3, 1.3, 1.3, 1.3, 1.3, 1.3, 1.3, 1.3, 1.3, 1.3, 1.3]
_IONIZATION_ENERGIES = [0.0, 13.6, 24.59, 5.39, 9.32, 8.3, 11.26, 14.53, 13.62, 17.42, 21.56, 5.14, 7.65, 5.99, 8.15, 10.49, 10.36, 12.97, 15.76, 4.34, 6.11, 6.56, 6.83, 6.75, 6.77, 7.43, 7.9, 7.88, 7.64, 7.73, 9.39, 6.0, 7.9, 9.79, 9.75, 11.81, 14.0, 4.18, 5.69, 6.22, 6.63, 6.76, 7.09, 7.28, 7.36, 7.46, 8.34, 7.58, 8.99, 5.79, 7.34, 8.64, 9.01, 10.45, 12.13, 3.89, 5.21, 5.58, 5.54, 5.47, 5.53, 5.58, 5.64, 5.67, 6.15, 5.86, 5.94, 6.02, 6.11, 6.18, 6.25, 5.43, 6.83, 7.55, 7.86, 7.83, 8.44, 8.97, 8.96, 9.23, 10.44, 6.11, 7.42, 7.29, 8.42, 9.3, 10.75, 4.07, 5.28, 5.17, 6.31, 5.89, 6.19, 6.27, 6.03, 5.97, 6.02, 6.2, 6.28, 6.42, 6.5, 6.58, 6.65, 4.9, 6.0, 6.0, 6.0, 6.0, 6.0, 6.0, 6.0, 6.0, 6.0, 6.0, 6.0, 6.0, 6.0, 6.0, 6.0]

_PROPS = np.zeros((119, 4), dtype=np.float32)
_PROPS[: len(_ATOMIC_MASSES), 0] = _ATOMIC_MASSES
_PROPS[: len(_ATOMIC_RADII), 1] = _ATOMIC_RADII
_PROPS[: len(_ELECTRONEGATIVITIES), 2] = _ELECTRONEGATIVITIES
_PROPS[: len(_IONIZATION_ENERGIES), 3] = _IONIZATION_ENERGIES

_N = 100000
_D = 128
_NW = 32
_CHUNK = 128
_K = 25
_PER_W = _CHUNK * _K
_NPAD = _NW * _PER_W


def _fuse_body(ct_ref, w_ref, b_ref, out_ref):
    out_ref[...] = (
        jnp.dot(ct_ref[...], w_ref[...], preferred_element_type=jnp.float32)
        + b_ref[...]
    )


def _fused_table(table, W, b):
    ct = jnp.zeros((128, 256), dtype=jnp.float32)
    ct = ct.at[:119, :_D].set(table)
    ct = ct.at[:119, _D : _D + 4].set(_PROPS)
    w_pad = jnp.zeros((256, _D), dtype=jnp.float32).at[: _D + 4, :].set(W)
    return pl.pallas_call(
        _fuse_body,
        out_shape=jax.ShapeDtypeStruct((128, _D), jnp.float32),
    )(ct, w_pad, b.reshape(1, _D))


_sc_mesh = plsc.VectorSubcoreMesh(core_axis_name="c", subcore_axis_name="s")

_NBUF = 5
_KOUT = _K // _NBUF


_LAST_W = _NW - 1
_K31 = (_N - _LAST_W * _PER_W) // _CHUNK
_TAIL = _N - _LAST_W * _PER_W - _K31 * _CHUNK


@functools.partial(
    pl.kernel,
    mesh=_sc_mesh,
    out_type=jax.ShapeDtypeStruct((_N, _D), jnp.float32),
    scratch_types=(
        [pltpu.VMEM((_K, _CHUNK), jnp.int32)]
        + [pltpu.VMEM_SHARED((128, _D), jnp.float32)]
        + [pltpu.VMEM((_TAIL, _D), jnp.float32)]
        + [pltpu.VMEM((_CHUNK, _D), jnp.float32)] * _NBUF
        + [pltpu.SemaphoreType.DMA] * (2 * _NBUF)
    ),
)
def _sc_gather(idx_hbm, fused_hbm, out_hbm, idx_v, shared_tab, tail_v, *scratch):
    bufs = scratch[:_NBUF]
    gsems = scratch[_NBUF : 2 * _NBUF]
    ssems = scratch[2 * _NBUF :]
    sid = lax.axis_index("s")
    wid = sid * 2 + lax.axis_index("c")
    base = wid * _PER_W
    pltpu.sync_copy(idx_hbm.at[wid], idx_v)

    @pl.when(sid == 0)
    def _():
        pltpu.sync_copy(fused_hbm, shared_tab)

    plsc.subcore_barrier()

    @pl.when(wid < _LAST_W)
    def _full_ring():
        for b in range(_NBUF):
            pltpu.async_copy(shared_tab.at[idx_v.at[b]], bufs[b], gsems[b])

        def outer(g, carry):
            for b in range(_NBUF):
                j = g * _NBUF + b
                pltpu.make_async_copy(
                    shared_tab.at[idx_v.at[j]], bufs[b], gsems[b]
                ).wait()
                sc = pltpu.make_async_copy(
                    bufs[b],
                    out_hbm.at[pl.ds(base + j * _CHUNK, _CHUNK)],
                    ssems[b],
                )
                sc.start()
                sc.wait()

                @pl.when(g < _KOUT - 1)
                def _():
                    pltpu.async_copy(
                        shared_tab.at[idx_v.at[j + _NBUF]], bufs[b], gsems[b]
                    )

            return carry

        lax.fori_loop(0, _KOUT, outer, 0)

    @pl.when(wid == _LAST_W)
    def _tail_worker():
        def body(j, carry):
            pltpu.async_copy(
                shared_tab.at[idx_v.at[j]], bufs[0], gsems[0]
            ).wait()
            pltpu.sync_copy(
                bufs[0], out_hbm.at[pl.ds(base + j * _CHUNK, _CHUNK)]
            )
            return carry

        lax.fori_loop(0, _K31, body, 0)
        pltpu.async_copy(
            shared_tab.at[idx_v.at[_K31, pl.ds(0, _TAIL)]], tail_v, gsems[1]
        ).wait()
        pltpu.sync_copy(
            tail_v, out_hbm.at[pl.ds(base + _K31 * _CHUNK, _TAIL)]
        )


def kernel(atomic_numbers, table, W, b):
    fused = _fused_table(table, W, b)
    an = atomic_numbers.astype(jnp.int32)
    an_pad = jnp.zeros((_NPAD,), jnp.int32).at[:_N].set(an)
    idx3 = an_pad.reshape(_NW, _K, _CHUNK)
    return _sc_gather(idx3, fused)


# speedup vs baseline: 67.8335x; 1.2667x over previous
import functools

import jax
import jax.numpy as jnp
import numpy as np
from jax import lax
from jax.experimental import pallas as pl
from jax.experimental.pallas import tpu as pltpu
from jax.experimental.pallas import tpu_sc as plsc

_ATOMIC_MASSES = [0.0, 1.008, 4.003, 6.941, 9.012, 10.81, 12.01, 14.01, 16.0, 19.0, 20.18, 22.99, 24.31, 26.98, 28.09, 30.97, 32.07, 35.45, 39.95, 39.1, 40.08, 44.96, 47.87, 50.94, 52.0, 54.94, 55.85, 58.93, 58.69, 63.55, 65.38, 69.72, 72.63, 74.92, 78.97, 79.9, 83.8, 85.47, 87.62, 88.91, 91.22, 92.91, 95.95, 98.0, 101.1, 102.9, 106.4, 107.9, 112.4, 114.8, 118.7, 121.8, 127.6, 126.9, 131.3, 132.9, 137.3, 138.9, 140.1, 140.9, 144.2, 145.0, 150.4, 152.0, 157.3, 158.9, 162.5, 164.9, 167.3, 168.9, 173.0, 175.0, 178.5, 180.9, 183.8, 186.2, 190.2, 192.2, 195.1, 197.0, 200.6, 204.4, 207.2, 209.0, 209.0, 210.0, 222.0, 223.0, 226.0, 227.0, 232.0, 231.0, 238.0, 237.0, 244.0, 243.0, 247.0, 247.0, 251.0, 252.0, 257.0, 258.0, 259.0, 262.0, 267.0, 270.0, 269.0, 270.0, 270.0, 278.0, 281.0, 281.0, 285.0, 286.0, 289.0, 289.0, 293.0, 293.0, 294.0]
_ATOMIC_RADII = [0.0, 1.2, 1.4, 1.82, 1.53, 1.92, 1.7, 1.55, 1.52, 1.47, 1.54, 2.27, 1.73, 1.84, 2.1, 1.8, 1.8, 1.75, 1.88, 2.75, 2.31, 2.11, 2.0, 2.0, 2.0, 2.0, 2.0, 2.0, 1.63, 1.4, 1.39, 1.87, 2.11, 1.85, 1.9, 1.85, 2.02, 3.03, 2.49, 2.0, 2.0, 2.0, 2.0, 2.0, 2.0, 2.0, 1.63, 1.72, 1.58, 1.93, 2.17, 2.06, 2.06, 1.98, 2.16, 3.43, 2.68, 2.0, 2.0, 2.0, 2.0, 2.0, 2.0, 2.0, 2.0, 2.0, 2.0, 2.0, 2.0, 2.0, 2.0, 2.0, 2.0, 2.0, 2.0, 2.0, 2.0, 2.0, 1.75, 1.66, 1.55, 1.96, 2.02, 2.07, 1.97, 2.02, 2.2, 3.48, 2.83, 2.0, 2.0, 2.0, 1.86, 2.0, 2.0, 2.0, 2.0, 2.0, 2.0, 2.0, 2.0, 2.0, 2.0, 2.0, 2.0, 2.0, 2.0, 2.0, 2.0, 2.0, 2.0, 2.0, 2.0, 2.0, 2.0, 2.0, 2.0, 2.0, 2.0]
_ELECTRONEGATIVITIES = [0.0, 2.2, 0.0, 0.98, 1.57, 2.04, 2.55, 3.04, 3.44, 3.98, 0.0, 0.93, 1.31, 1.61, 1.9, 2.19, 2.58, 3.16, 0.0, 0.82, 1.0, 1.36, 1.54, 1.63, 1.66, 1.55, 1.83, 1.88, 1.91, 1.9, 1.65, 1.81, 2.01, 2.18, 2.55, 2.96, 3.0, 0.82, 0.95, 1.22, 1.33, 1.6, 2.16, 1.9, 2.2, 2.28, 2.2, 1.93, 1.69, 1.78, 1.96, 2.05, 2.1, 2.66, 2.6, 0.79, 0.89, 1.1, 1.12, 1.13, 1.14, 1.13, 1.17, 1.2, 1.2, 1.22, 1.23, 1.24, 1.25, 1.1, 1.27, 1.3, 1.5, 2.36, 1.9, 2.2, 2.2, 2.28, 2.54, 2.0, 1.62, 1.87, 2.33, 2.02, 2.0, 2.2, 2.2, 0.7, 0.9, 1.1, 1.3, 1.5, 1.38, 1.36, 1.28, 1.3, 1.3, 1.3, 1.3, 1.3, 1.3, 1.3, 1.3, 1.3, 1.3, 1.3, 1.3, 1.3, 1.3, 1.3, 1.3, 1.3, 1.3, 1.3, 1.3, 1.3, 1.3, 1.3, 1.3]
_IONIZATION_ENERGIES = [0.0, 13.6, 24.59, 5.39, 9.32, 8.3, 11.26, 14.53, 13.62, 17.42, 21.56, 5.14, 7.65, 5.99, 8.15, 10.49, 10.36, 12.97, 15.76, 4.34, 6.11, 6.56, 6.83, 6.75, 6.77, 7.43, 7.9, 7.88, 7.64, 7.73, 9.39, 6.0, 7.9, 9.79, 9.75, 11.81, 14.0, 4.18, 5.69, 6.22, 6.63, 6.76, 7.09, 7.28, 7.36, 7.46, 8.34, 7.58, 8.99, 5.79, 7.34, 8.64, 9.01, 10.45, 12.13, 3.89, 5.21, 5.58, 5.54, 5.47, 5.53, 5.58, 5.64, 5.67, 6.15, 5.86, 5.94, 6.02, 6.11, 6.18, 6.25, 5.43, 6.83, 7.55, 7.86, 7.83, 8.44, 8.97, 8.96, 9.23, 10.44, 6.11, 7.42, 7.29, 8.42, 9.3, 10.75, 4.07, 5.28, 5.17, 6.31, 5.89, 6.19, 6.27, 6.03, 5.97, 6.02, 6.2, 6.28, 6.42, 6.5, 6.58, 6.65, 4.9, 6.0, 6.0, 6.0, 6.0, 6.0, 6.0, 6.0, 6.0, 6.0, 6.0, 6.0, 6.0, 6.0, 6.0, 6.0]

_PROPS = np.zeros((119, 4), dtype=np.float32)
_PROPS[: len(_ATOMIC_MASSES), 0] = _ATOMIC_MASSES
_PROPS[: len(_ATOMIC_RADII), 1] = _ATOMIC_RADII
_PROPS[: len(_ELECTRONEGATIVITIES), 2] = _ELECTRONEGATIVITIES
_PROPS[: len(_IONIZATION_ENERGIES), 3] = _IONIZATION_ENERGIES

_N = 100000
_D = 128
_NW = 32
_CHUNK = 128
_K = 25
_PER_W = _CHUNK * _K
_NPAD = _NW * _PER_W


_PROPS_PAD = np.zeros((128, 4), dtype=np.float32)
_PROPS_PAD[:119] = _PROPS


def _fuse_body(tab_ref, w_ref, b_ref, props_ref, out_ref):
    tabp = jnp.concatenate(
        [tab_ref[...], jnp.zeros((128 - 119, _D), jnp.float32)], axis=0
    )
    acc = jnp.dot(tabp, w_ref[0:_D, :], preferred_element_type=jnp.float32)
    acc = acc + jnp.dot(
        props_ref[...],
        w_ref[_D : _D + 4, :],
        preferred_element_type=jnp.float32,
    )
    out_ref[...] = acc + b_ref[...]


def _fused_table(table, W, b):
    return pl.pallas_call(
        _fuse_body,
        out_shape=jax.ShapeDtypeStruct((128, _D), jnp.float32),
    )(table, W, b.reshape(1, _D), jnp.asarray(_PROPS_PAD))


_sc_mesh = plsc.VectorSubcoreMesh(core_axis_name="c", subcore_axis_name="s")

_NBUF = 5
_KOUT = _K // _NBUF


_LAST_W = _NW - 1
_K31 = (_N - _LAST_W * _PER_W) // _CHUNK
_TAIL = _N - _LAST_W * _PER_W - _K31 * _CHUNK


@functools.partial(
    pl.kernel,
    mesh=_sc_mesh,
    out_type=jax.ShapeDtypeStruct((_N, _D), jnp.float32),
    scratch_types=(
        [pltpu.VMEM((_PER_W,), jnp.int32)]
        + [pltpu.VMEM_SHARED((128, _D), jnp.float32)]
        + [pltpu.VMEM((_TAIL, _D), jnp.float32)]
        + [pltpu.VMEM((_CHUNK, _D), jnp.float32)] * _NBUF
        + [pltpu.SemaphoreType.DMA] * (2 * _NBUF)
    ),
)
def _sc_gather(idx_hbm, fused_hbm, out_hbm, idx_v, shared_tab, tail_v, *scratch):
    bufs = scratch[:_NBUF]
    gsems = scratch[_NBUF : 2 * _NBUF]
    ssems = scratch[2 * _NBUF :]
    sid = lax.axis_index("s")
    wid = sid * 2 + lax.axis_index("c")
    base = wid * _PER_W

    @pl.when(wid < _LAST_W)
    def _stage_idx_full():
        pltpu.sync_copy(idx_hbm.at[pl.ds(base, _PER_W)], idx_v)

    @pl.when(wid == _LAST_W)
    def _stage_idx_tail():
        pltpu.sync_copy(
            idx_hbm.at[pl.ds(base, _K31 * _CHUNK + _TAIL)],
            idx_v.at[pl.ds(0, _K31 * _CHUNK + _TAIL)],
        )

    @pl.when(sid == 0)
    def _():
        pltpu.sync_copy(fused_hbm, shared_tab)

    plsc.subcore_barrier()

    @pl.when(wid < _LAST_W)
    def _full_ring():
        for b in range(_NBUF):
            pltpu.async_copy(shared_tab.at[idx_v.at[pl.ds(b * _CHUNK, _CHUNK)]], bufs[b], gsems[b])

        def outer(g, carry):
            for b in range(_NBUF):
                j = g * _NBUF + b
                pltpu.make_async_copy(
                    shared_tab.at[idx_v.at[pl.ds(j * _CHUNK, _CHUNK)]], bufs[b], gsems[b]
                ).wait()
                sc = pltpu.make_async_copy(
                    bufs[b],
                    out_hbm.at[pl.ds(base + j * _CHUNK, _CHUNK)],
                    ssems[b],
                )
                sc.start()
                sc.wait()

                @pl.when(g < _KOUT - 1)
                def _():
                    pltpu.async_copy(
                        shared_tab.at[idx_v.at[pl.ds((j + _NBUF) * _CHUNK, _CHUNK)]], bufs[b], gsems[b]
                    )

            return carry

        lax.fori_loop(0, _KOUT, outer, 0)

    @pl.when(wid == _LAST_W)
    def _tail_worker():
        def body(j, carry):
            pltpu.async_copy(
                shared_tab.at[idx_v.at[pl.ds(j * _CHUNK, _CHUNK)]], bufs[0], gsems[0]
            ).wait()
            pltpu.sync_copy(
                bufs[0], out_hbm.at[pl.ds(base + j * _CHUNK, _CHUNK)]
            )
            return carry

        lax.fori_loop(0, _K31, body, 0)
        pltpu.async_copy(
            shared_tab.at[idx_v.at[pl.ds(_K31 * _CHUNK, _TAIL)]], tail_v, gsems[1]
        ).wait()
        pltpu.sync_copy(
            tail_v, out_hbm.at[pl.ds(base + _K31 * _CHUNK, _TAIL)]
        )


def kernel(atomic_numbers, table, W, b):
    fused = _fused_table(table, W, b)
    an = atomic_numbers.astype(jnp.int32)
    return _sc_gather(an, fused)
